# Initial kernel scaffold; baseline (speedup 1.0000x reference)
import jax
import jax.numpy as jnp
from jax.experimental import pallas as pl


def kernel(x, id_index, edge_index, cluster, batch, id_emb, enc_w1, enc_b1, enc_w2, enc_b2, enc_w3, enc_b3, skip_w, skip_b, conv_W, conv_b, ln_g, ln_b):
    raise NotImplementedError("write your pallas kernel here")


# speedup vs baseline: 5.3720x; 5.3720x over previous
import functools

import jax
import jax.numpy as jnp
from jax import lax
from jax.experimental import pallas as pl
from jax.experimental.pallas import tpu as pltpu
from jax.experimental.pallas import tpu_sc as plsc

N = 10000
E = 160000
HID = 256
NC = 500
NLAYERS = 9

NSC = 2
NTILES = 16
GROUP = 128
NGROUPS = 1280
PE = NGROUPS * GROUP
GPT16 = NGROUPS // NTILES
GPT32 = NGROUPS // (NTILES * NSC)
ACC_ROWS = 10112
SLAB = ACC_ROWS // NTILES
OUT_LAST = N - (NTILES - 1) * SLAB
HALF = HID // 2

BLK = 2000
NG = N // BLK

def _copy_out(acc, out_hbm, c, s, width):
    @pl.when(s < NTILES - 1)
    def _():
        pltpu.sync_copy(acc.at[pl.ds(s * SLAB, SLAB)],
                        out_hbm.at[c, pl.ds(s * SLAB, SLAB)])

    @pl.when(s == NTILES - 1)
    def _():
        base = (NTILES - 1) * SLAB
        pltpu.sync_copy(acc.at[pl.ds(base, OUT_LAST)],
                        out_hbm.at[c, pl.ds(base, OUT_LAST)])


def _sc_degree_body(dst_hbm, ones_hbm, zeros_hbm, out_hbm, didx, ones_v, acc):
    c = lax.axis_index("c")
    s = lax.axis_index("s")
    pltpu.sync_copy(zeros_hbm.at[pl.ds(s * SLAB, SLAB)],
                    acc.at[pl.ds(s * SLAB, SLAB)])
    pltpu.sync_copy(ones_hbm, ones_v)
    g0 = (s * NSC + c) * GPT32
    pltpu.sync_copy(dst_hbm.at[pl.ds(g0, GPT32)], didx)
    plsc.subcore_barrier()

    def body(g, carry):
        pltpu.sync_copy(ones_v, acc.at[didx.at[g]], add=True)
        return carry

    lax.fori_loop(0, GPT32, body, 0)
    plsc.subcore_barrier()
    _copy_out(acc, out_hbm, c, s, 16)


def _sc_edge_scatter_body(y_hbm, src_hbm, dst_hbm, zeros_hbm, out_hbm,
                          sidx, didx, rows, acc, sem):
    c = lax.axis_index("c")
    s = lax.axis_index("s")
    pltpu.sync_copy(zeros_hbm.at[pl.ds(s * SLAB, SLAB)],
                    acc.at[pl.ds(s * SLAB, SLAB)])
    g0 = s * GPT16
    pltpu.sync_copy(src_hbm.at[c, pl.ds(g0, GPT16)], sidx)
    pltpu.sync_copy(dst_hbm.at[pl.ds(g0, GPT16)], didx)
    plsc.subcore_barrier()

    def body(g, carry):
        pltpu.async_copy(y_hbm.at[sidx.at[g]], rows, sem).wait()
        pltpu.sync_copy(rows, acc.at[didx.at[g]], add=True)
        return carry

    lax.fori_loop(0, GPT16, body, 0)
    plsc.subcore_barrier()
    _copy_out(acc, out_hbm, c, s, HALF)


@functools.cache
def _sc_kernels():
    mesh = plsc.VectorSubcoreMesh(core_axis_name="c", subcore_axis_name="s",
                                  num_cores=NSC, num_subcores=NTILES)
    degree = pl.kernel(
        _sc_degree_body,
        out_type=jax.ShapeDtypeStruct((NSC, N, 16), jnp.float32),
        mesh=mesh,
        scratch_types=[
            pltpu.VMEM((GPT32, GROUP), jnp.int32),
            pltpu.VMEM((GROUP, 16), jnp.float32),
            pltpu.VMEM_SHARED((ACC_ROWS, 16), jnp.float32),
        ],
    )
    edge_scatter = pl.kernel(
        _sc_edge_scatter_body,
        out_type=jax.ShapeDtypeStruct((NSC, N, HALF), jnp.float32),
        mesh=mesh,
        scratch_types=[
            pltpu.VMEM((GPT16, GROUP), jnp.int32),
            pltpu.VMEM((GPT16, GROUP), jnp.int32),
            pltpu.VMEM((GROUP, HALF), jnp.float32),
            pltpu.VMEM_SHARED((ACC_ROWS, HALF), jnp.float32),
            pltpu.SemaphoreType.DMA,
        ],
    )
    return degree, edge_scatter


_SQRT2 = 1.4142135623730951


def _gelu(x):
    return 0.5 * x * (1.0 + lax.erf(x / _SQRT2))


def _dis_of(degp):
    return lax.rsqrt(degp[0, :, :1] + degp[1, :, :1] + 1.0)


def _enc_body(x_ref, id_ref, degp_ref, emb_ref, w1_ref, b1_ref, w2_ref,
              b2_ref, w3_ref, b3_ref, wsk_ref, bsk_ref, w0_ref,
              xx_ref, y_ref):
    ids = id_ref[...]
    oh = (ids == lax.broadcasted_iota(jnp.int32, (BLK, 65), 1))
    oh = oh.astype(jnp.float32)
    emb = emb_ref[...]
    x = x_ref[...]
    w1 = w1_ref[...]
    wsk = wsk_ref[...]
    h = x @ w1[0:3, :] + oh @ (emb @ w1[3:11, :]) + b1_ref[...]
    h = _gelu(h)
    h = _gelu(h @ w2_ref[...] + b2_ref[...])
    h = h @ w3_ref[...] + b3_ref[...]
    sk = x @ wsk[0:3, :] + oh @ (emb @ wsk[3:11, :]) + bsk_ref[...]
    xx = _gelu(sk + h)
    xx_ref[...] = xx
    dis = _dis_of(degp_ref[...])
    y = (xx * dis) @ w0_ref[...]
    y_ref[0] = y[:, :HALF]
    y_ref[1] = y[:, HALF:]


def _epilogue(acc, y, degp, b, g, bn):
    a = jnp.concatenate([acc[0] + y[0], acc[1] + y[1]], axis=1)
    dis = _dis_of(degp)
    z = a * dis + b
    mu = jnp.mean(z, axis=1, keepdims=True)
    var = jnp.mean((z - mu) ** 2, axis=1, keepdims=True)
    z = (z - mu) * lax.rsqrt(var + 1e-5) * g + bn
    return jnp.maximum(z, 0.0), dis


def _layer_body(add_orig, *refs):
    if add_orig:
        (acc_ref, y_ref, degp_ref, b_ref, g_ref, bn_ref, wn_ref, orig_ref,
         out_ref) = refs
    else:
        acc_ref, y_ref, degp_ref, b_ref, g_ref, bn_ref, wn_ref, out_ref = refs
    z, dis = _epilogue(acc_ref[...], y_ref[...], degp_ref[...],
                       b_ref[...], g_ref[...], bn_ref[...])
    if add_orig:
        z = z + orig_ref[...]
    yn = (z * dis) @ wn_ref[...]
    out_ref[0] = yn[:, :HALF]
    out_ref[1] = yn[:, HALF:]


def _final_body(acc_ref, y_ref, degp_ref, b_ref, g_ref, bn_ref, cl_ref,
                out_ref, scr_ref):
    i = pl.program_id(0)
    z, _ = _epilogue(acc_ref[...], y_ref[...], degp_ref[...],
                     b_ref[...], g_ref[...], bn_ref[...])
    cl = cl_ref[...]
    oh = (cl == lax.broadcasted_iota(jnp.int32, (BLK, 512), 1))
    oh = oh.astype(jnp.float32)
    rhs = jnp.concatenate([z, jnp.ones((BLK, HALF), jnp.float32)], axis=1)
    part = lax.dot_general(oh, rhs, (((0,), (0,)), ((), ())))

    @pl.when(i == 0)
    def _():
        scr_ref[...] = part

    @pl.when(i > 0)
    def _():
        scr_ref[...] = scr_ref[...] + part

    @pl.when(i == NG - 1)
    def _():
        agg = scr_ref[...]
        sums = agg[:, :HID]
        cnt = agg[:, HID:HID + 1]
        pooled = sums / jnp.maximum(cnt, 1.0)
        nrm = jnp.sqrt(jnp.sum(pooled * pooled, axis=0, keepdims=True))
        out_ref[...] = pooled / jnp.maximum(nrm, 1e-6)


_spec_rows2 = pl.BlockSpec((2, BLK, HALF), lambda i: (0, i, 0))
_spec_deg = pl.BlockSpec((2, BLK, 16), lambda i: (0, i, 0))
_spec_full = lambda *shape: pl.BlockSpec(shape, lambda i: (0,) * len(shape))
_spec_rows = lambda m: pl.BlockSpec((BLK, m), lambda i: (i, 0))

_f32 = jnp.float32


def kernel(x, id_index, edge_index, cluster, batch, id_emb, enc_w1, enc_b1,
           enc_w2, enc_b2, enc_w3, enc_b3, skip_w, skip_b, conv_W, conv_b,
           ln_g, ln_b):
    src = edge_index[0].astype(jnp.int32)
    dst = edge_index[1].astype(jnp.int32)
    srcp = jnp.concatenate([src, jnp.zeros((PE - E,), jnp.int32)])
    dstp = jnp.concatenate([dst, jnp.full((PE - E,), N, jnp.int32)])
    src3 = jnp.stack([srcp, srcp + N]).reshape(NSC, NGROUPS, GROUP)
    dst3 = dstp.reshape(NGROUPS, GROUP)
    zeros128 = jnp.zeros((ACC_ROWS, HALF), _f32)
    zeros16 = jnp.zeros((ACC_ROWS, 16), _f32)
    ones16 = jnp.ones((GROUP, 16), _f32)
    idc = id_index.astype(jnp.int32).reshape(N, 1)
    clc = cluster.astype(jnp.int32).reshape(N, 1)

    sc_degree, sc_edge_scatter = _sc_kernels()
    degp = sc_degree(dst3, ones16, zeros16)

    enc = pl.pallas_call(
        _enc_body,
        grid=(NG,),
        in_specs=[
            _spec_rows(3), _spec_rows(1), _spec_deg,
            _spec_full(65, 8),
            _spec_full(11, 64), _spec_full(1, 64),
            _spec_full(64, 64), _spec_full(1, 64),
            _spec_full(64, HID), _spec_full(1, HID),
            _spec_full(11, HID), _spec_full(1, HID),
            _spec_full(HID, HID),
        ],
        out_specs=[_spec_rows(HID), _spec_rows2],
        out_shape=[jax.ShapeDtypeStruct((N, HID), _f32),
                   jax.ShapeDtypeStruct((NSC, N, HALF), _f32)],
    )
    xx, y = enc(x, idc, degp, id_emb,
                enc_w1, enc_b1.reshape(1, -1),
                enc_w2, enc_b2.reshape(1, -1),
                enc_w3, enc_b3.reshape(1, -1),
                skip_w, skip_b.reshape(1, -1),
                conv_W[0])

    for i in range(NLAYERS - 1):
        acc = sc_edge_scatter(y.reshape(NSC * N, HALF), src3, dst3, zeros128)
        add_orig = (i == 0)
        in_specs = [
            _spec_rows2, _spec_rows2, _spec_deg,
            _spec_full(1, HID), _spec_full(1, HID), _spec_full(1, HID),
            _spec_full(HID, HID),
        ]
        args = [acc, y, degp,
                conv_b[i].reshape(1, -1), ln_g[i].reshape(1, -1),
                ln_b[i].reshape(1, -1), conv_W[i + 1]]
        if add_orig:
            in_specs.append(_spec_rows(HID))
            args.append(xx)
        y = pl.pallas_call(
            functools.partial(_layer_body, add_orig),
            grid=(NG,),
            in_specs=in_specs,
            out_specs=_spec_rows2,
            out_shape=jax.ShapeDtypeStruct((NSC, N, HALF), _f32),
        )(*args)

    acc = sc_edge_scatter(y.reshape(NSC * N, HALF), src3, dst3, zeros128)
    i = NLAYERS - 1
    res = pl.pallas_call(
        _final_body,
        grid=(NG,),
        in_specs=[
            _spec_rows2, _spec_rows2, _spec_deg,
            _spec_full(1, HID), _spec_full(1, HID), _spec_full(1, HID),
            _spec_rows(1),
        ],
        out_specs=pl.BlockSpec((512, HID), lambda i: (0, 0)),
        out_shape=jax.ShapeDtypeStruct((512, HID), _f32),
        scratch_shapes=[pltpu.VMEM((512, HID + HALF), _f32)],
    )(acc, y, degp,
      conv_b[i].reshape(1, -1), ln_g[i].reshape(1, -1),
      ln_b[i].reshape(1, -1), clc)
    return res[:NC]


# speedup vs baseline: 5.8006x; 1.0798x over previous
import functools

import jax
import jax.numpy as jnp
from jax import lax
from jax.experimental import pallas as pl
from jax.experimental.pallas import tpu as pltpu
from jax.experimental.pallas import tpu_sc as plsc

N = 10000
E = 160000
HID = 256
NC = 500
NLAYERS = 9

NSC = 2
NTILES = 16
GROUP = 128
NGROUPS = 1280
PE = NGROUPS * GROUP
GPT16 = NGROUPS // NTILES
GPHASE = GPT16 // 2
GPT32 = NGROUPS // (NTILES * NSC)
ACC_ROWS = 10112
SLAB = ACC_ROWS // NTILES
OUT_LAST = N - (NTILES - 1) * SLAB
HALF = HID // 2

BLK = 2000
NG = N // BLK

def _copy_out(acc, out_hbm, c, s, width):
    @pl.when(s < NTILES - 1)
    def _():
        pltpu.sync_copy(acc.at[pl.ds(s * SLAB, SLAB)],
                        out_hbm.at[c, pl.ds(s * SLAB, SLAB)])

    @pl.when(s == NTILES - 1)
    def _():
        base = (NTILES - 1) * SLAB
        pltpu.sync_copy(acc.at[pl.ds(base, OUT_LAST)],
                        out_hbm.at[c, pl.ds(base, OUT_LAST)])


def _sc_degree_body(dst_hbm, ones_hbm, zeros_hbm, out_hbm, didx, ones_v, acc):
    c = lax.axis_index("c")
    s = lax.axis_index("s")
    pltpu.sync_copy(zeros_hbm.at[pl.ds(s * SLAB, SLAB)],
                    acc.at[pl.ds(s * SLAB, SLAB)])
    pltpu.sync_copy(ones_hbm, ones_v)
    g0 = (s * NSC + c) * GPT32
    pltpu.sync_copy(dst_hbm.at[pl.ds(g0, GPT32)], didx)
    plsc.subcore_barrier()

    def body(g, carry):
        pltpu.sync_copy(ones_v, acc.at[didx.at[g]], add=True)
        return carry

    lax.fori_loop(0, GPT32, body, 0)
    plsc.subcore_barrier()
    _copy_out(acc, out_hbm, c, s, 16)


def _sc_edge_scatter_body(y_hbm, src_hbm, dst_hbm, zeros_hbm, out_hbm,
                          sidx, didx, rows0, rows1, acc, sem):
    c = lax.axis_index("c")
    s = lax.axis_index("s")
    pltpu.sync_copy(zeros_hbm.at[pl.ds(s * SLAB, SLAB)],
                    acc.at[pl.ds(s * SLAB, SLAB)])
    plsc.subcore_barrier()
    for p in range(2):
        g0 = s * GPT16 + p * GPHASE
        pltpu.sync_copy(src_hbm.at[c, pl.ds(g0, GPHASE)], sidx)
        pltpu.sync_copy(dst_hbm.at[pl.ds(g0, GPHASE)], didx)

        pltpu.async_copy(y_hbm.at[sidx.at[0]], rows0, sem)

        def body(h, carry):
            g = 2 * h
            pltpu.make_async_copy(y_hbm.at[sidx.at[g]], rows0, sem).wait()
            pltpu.async_copy(y_hbm.at[sidx.at[g + 1]], rows1, sem)
            pltpu.sync_copy(rows0, acc.at[didx.at[g]], add=True)
            pltpu.make_async_copy(y_hbm.at[sidx.at[g + 1]], rows1, sem).wait()
            pltpu.async_copy(y_hbm.at[sidx.at[g + 2]], rows0, sem)
            pltpu.sync_copy(rows1, acc.at[didx.at[g + 1]], add=True)
            return carry

        lax.fori_loop(0, GPHASE // 2 - 1, body, 0)
        g = GPHASE - 2
        pltpu.make_async_copy(y_hbm.at[sidx.at[g]], rows0, sem).wait()
        pltpu.async_copy(y_hbm.at[sidx.at[g + 1]], rows1, sem)
        pltpu.sync_copy(rows0, acc.at[didx.at[g]], add=True)
        pltpu.make_async_copy(y_hbm.at[sidx.at[g + 1]], rows1, sem).wait()
        pltpu.sync_copy(rows1, acc.at[didx.at[g + 1]], add=True)
    plsc.subcore_barrier()
    _copy_out(acc, out_hbm, c, s, HALF)


@functools.cache
def _sc_kernels():
    mesh = plsc.VectorSubcoreMesh(core_axis_name="c", subcore_axis_name="s",
                                  num_cores=NSC, num_subcores=NTILES)
    degree = pl.kernel(
        _sc_degree_body,
        out_type=jax.ShapeDtypeStruct((NSC, N, 16), jnp.float32),
        mesh=mesh,
        scratch_types=[
            pltpu.VMEM((GPT32, GROUP), jnp.int32),
            pltpu.VMEM((GROUP, 16), jnp.float32),
            pltpu.VMEM_SHARED((ACC_ROWS, 16), jnp.float32),
        ],
    )
    edge_scatter = pl.kernel(
        _sc_edge_scatter_body,
        out_type=jax.ShapeDtypeStruct((NSC, N, HALF), jnp.float32),
        mesh=mesh,
        scratch_types=[
            pltpu.VMEM((GPHASE, GROUP), jnp.int32),
            pltpu.VMEM((GPHASE, GROUP), jnp.int32),
            pltpu.VMEM((GROUP, HALF), jnp.float32),
            pltpu.VMEM((GROUP, HALF), jnp.float32),
            pltpu.VMEM_SHARED((ACC_ROWS, HALF), jnp.float32),
            pltpu.SemaphoreType.DMA,
        ],
    )
    return degree, edge_scatter


_SQRT2 = 1.4142135623730951


def _gelu(x):
    return 0.5 * x * (1.0 + lax.erf(x / _SQRT2))


def _dis_of(dis_ref):
    return dis_ref[:, :1]


def _enc_body(x_ref, id_ref, degp_ref, emb_ref, w1_ref, b1_ref, w2_ref,
              b2_ref, w3_ref, b3_ref, wsk_ref, bsk_ref, w0_ref,
              xx_ref, y_ref):
    ids = id_ref[...]
    oh = (ids == lax.broadcasted_iota(jnp.int32, (BLK, 65), 1))
    oh = oh.astype(jnp.float32)
    emb = emb_ref[...]
    x = x_ref[...]
    w1 = w1_ref[...]
    wsk = wsk_ref[...]
    h = x @ w1[0:3, :] + oh @ (emb @ w1[3:11, :]) + b1_ref[...]
    h = _gelu(h)
    h = _gelu(h @ w2_ref[...] + b2_ref[...])
    h = h @ w3_ref[...] + b3_ref[...]
    sk = x @ wsk[0:3, :] + oh @ (emb @ wsk[3:11, :]) + bsk_ref[...]
    xx = _gelu(sk + h)
    xx_ref[...] = xx
    dis = _dis_of(degp_ref[...])
    y = (xx * dis) @ w0_ref[...]
    y_ref[0] = y[:, :HALF]
    y_ref[1] = y[:, HALF:]


def _epilogue(acc, y, degp, b, g, bn):
    a = jnp.concatenate([acc[0] + y[0], acc[1] + y[1]], axis=1)
    dis = _dis_of(degp)
    z = a * dis + b
    mu = jnp.mean(z, axis=1, keepdims=True)
    var = jnp.mean((z - mu) ** 2, axis=1, keepdims=True)
    z = (z - mu) / jnp.sqrt(var + 1e-5) * g + bn
    return jnp.maximum(z, 0.0), dis


def _layer_body(add_orig, *refs):
    if add_orig:
        (acc_ref, y_ref, degp_ref, b_ref, g_ref, bn_ref, wn_ref, orig_ref,
         out_ref) = refs
    else:
        acc_ref, y_ref, degp_ref, b_ref, g_ref, bn_ref, wn_ref, out_ref = refs
    z, dis = _epilogue(acc_ref[...], y_ref[...], degp_ref[...],
                       b_ref[...], g_ref[...], bn_ref[...])
    if add_orig:
        z = z + orig_ref[...]
    yn = (z * dis) @ wn_ref[...]
    out_ref[0] = yn[:, :HALF]
    out_ref[1] = yn[:, HALF:]


def _final_body(acc_ref, y_ref, degp_ref, b_ref, g_ref, bn_ref, cl_ref,
                out_ref, scr_ref):
    i = pl.program_id(0)
    z, _ = _epilogue(acc_ref[...], y_ref[...], degp_ref[...],
                     b_ref[...], g_ref[...], bn_ref[...])
    cl = cl_ref[...]
    oh = (cl == lax.broadcasted_iota(jnp.int32, (BLK, 512), 1))
    oh = oh.astype(jnp.float32)
    rhs = jnp.concatenate([z, jnp.ones((BLK, HALF), jnp.float32)], axis=1)
    part = lax.dot_general(oh, rhs, (((0,), (0,)), ((), ())))

    @pl.when(i == 0)
    def _():
        scr_ref[...] = part

    @pl.when(i > 0)
    def _():
        scr_ref[...] = scr_ref[...] + part

    @pl.when(i == NG - 1)
    def _():
        agg = scr_ref[...]
        sums = agg[:, :HID]
        cnt = agg[:, HID:HID + 1]
        pooled = sums / jnp.maximum(cnt, 1.0)
        nrm = jnp.sqrt(jnp.sum(pooled * pooled, axis=0, keepdims=True))
        out_ref[...] = pooled / jnp.maximum(nrm, 1e-6)


_spec_rows2 = pl.BlockSpec((2, BLK, HALF), lambda i: (0, i, 0))
_spec_deg = pl.BlockSpec((BLK, 16), lambda i: (i, 0))
_spec_full = lambda *shape: pl.BlockSpec(shape, lambda i: (0,) * len(shape))
_spec_rows = lambda m: pl.BlockSpec((BLK, m), lambda i: (i, 0))

_f32 = jnp.float32


def kernel(x, id_index, edge_index, cluster, batch, id_emb, enc_w1, enc_b1,
           enc_w2, enc_b2, enc_w3, enc_b3, skip_w, skip_b, conv_W, conv_b,
           ln_g, ln_b):
    src = edge_index[0].astype(jnp.int32)
    dst = edge_index[1].astype(jnp.int32)
    srcp = jnp.concatenate([src, jnp.zeros((PE - E,), jnp.int32)])
    dstp = jnp.concatenate([dst, jnp.full((PE - E,), N, jnp.int32)])
    src3 = jnp.stack([srcp, srcp + N]).reshape(NSC, NGROUPS, GROUP)
    dst3 = dstp.reshape(NGROUPS, GROUP)
    zeros128 = jnp.zeros((ACC_ROWS, HALF), _f32)
    zeros16 = jnp.zeros((ACC_ROWS, 16), _f32)
    ones16 = jnp.ones((GROUP, 16), _f32)
    idc = id_index.astype(jnp.int32).reshape(N, 1)
    clc = cluster.astype(jnp.int32).reshape(N, 1)

    sc_degree, sc_edge_scatter = _sc_kernels()
    degp = sc_degree(dst3, ones16, zeros16)
    deg = degp[0, :, 0] + degp[1, :, 0] + 1.0
    dis = jnp.where(deg > 0, deg ** -0.5, 0.0)
    degp = jnp.broadcast_to(dis[:, None], (N, 16))

    enc = pl.pallas_call(
        _enc_body,
        grid=(NG,),
        in_specs=[
            _spec_rows(3), _spec_rows(1), _spec_deg,
            _spec_full(65, 8),
            _spec_full(11, 64), _spec_full(1, 64),
            _spec_full(64, 64), _spec_full(1, 64),
            _spec_full(64, HID), _spec_full(1, HID),
            _spec_full(11, HID), _spec_full(1, HID),
            _spec_full(HID, HID),
        ],
        out_specs=[_spec_rows(HID), _spec_rows2],
        out_shape=[jax.ShapeDtypeStruct((N, HID), _f32),
                   jax.ShapeDtypeStruct((NSC, N, HALF), _f32)],
    )
    xx, y = enc(x, idc, degp, id_emb,
                enc_w1, enc_b1.reshape(1, -1),
                enc_w2, enc_b2.reshape(1, -1),
                enc_w3, enc_b3.reshape(1, -1),
                skip_w, skip_b.reshape(1, -1),
                conv_W[0])

    for i in range(NLAYERS - 1):
        acc = sc_edge_scatter(y.reshape(NSC * N, HALF), src3, dst3, zeros128)
        add_orig = (i == 0)
        in_specs = [
            _spec_rows2, _spec_rows2, _spec_deg,
            _spec_full(1, HID), _spec_full(1, HID), _spec_full(1, HID),
            _spec_full(HID, HID),
        ]
        args = [acc, y, degp,
                conv_b[i].reshape(1, -1), ln_g[i].reshape(1, -1),
                ln_b[i].reshape(1, -1), conv_W[i + 1]]
        if add_orig:
            in_specs.append(_spec_rows(HID))
            args.append(xx)
        y = pl.pallas_call(
            functools.partial(_layer_body, add_orig),
            grid=(NG,),
            in_specs=in_specs,
            out_specs=_spec_rows2,
            out_shape=jax.ShapeDtypeStruct((NSC, N, HALF), _f32),
        )(*args)

    acc = sc_edge_scatter(y.reshape(NSC * N, HALF), src3, dst3, zeros128)
    i = NLAYERS - 1
    res = pl.pallas_call(
        _final_body,
        grid=(NG,),
        in_specs=[
            _spec_rows2, _spec_rows2, _spec_deg,
            _spec_full(1, HID), _spec_full(1, HID), _spec_full(1, HID),
            _spec_rows(1),
        ],
        out_specs=pl.BlockSpec((512, HID), lambda i: (0, 0)),
        out_shape=jax.ShapeDtypeStruct((512, HID), _f32),
        scratch_shapes=[pltpu.VMEM((512, HID + HALF), _f32)],
    )(acc, y, degp,
      conv_b[i].reshape(1, -1), ln_g[i].reshape(1, -1),
      ln_b[i].reshape(1, -1), clc)
    return res[:NC]
